# baseline (device time: 206421 ns/iter reference)
import jax
import jax.numpy as jnp
from jax import lax
from jax.experimental import pallas as pl
from jax.experimental.pallas import tpu as pltpu

N_DEV = 16


def kernel(x, w_mat):
    m, _ = x.shape
    _, n = w_mat.shape
    chunk = m // N_DEV

    def body(x_ref, w_ref, out_ref, comm_ref, send_sems, recv_sems):
        my = lax.axis_index("i")
        right = lax.rem(my + 1, N_DEV)
        left = lax.rem(my + N_DEV - 1, N_DEV)

        barrier = pltpu.get_barrier_semaphore()
        for nbr in (left, right):
            pl.semaphore_signal(
                barrier, inc=1,
                device_id=(nbr,), device_id_type=pl.DeviceIdType.MESH,
            )
        pl.semaphore_wait(barrier, 2)

        def partial_chunk(idx):
            x_sub = x_ref[pl.ds(idx * chunk, chunk), :]
            return jnp.dot(x_sub, w_ref[:, :], preferred_element_type=jnp.float32)

        comm_ref[0] = partial_chunk(lax.rem(my + N_DEV - 1, N_DEV))

        for s in range(N_DEV - 1):
            rdma = pltpu.make_async_remote_copy(
                src_ref=comm_ref.at[s],
                dst_ref=comm_ref.at[s + 1],
                send_sem=send_sems.at[s],
                recv_sem=recv_sems.at[s],
                device_id=(right,),
                device_id_type=pl.DeviceIdType.MESH,
            )
            rdma.start()
            rdma.wait()

            idx = lax.rem(my + (N_DEV - 2 - s), N_DEV)
            p = partial_chunk(idx)
            if s < N_DEV - 2:
                comm_ref[s + 1] = comm_ref[s + 1] + p
            else:
                y = comm_ref[s + 1] + p
                c = 0.7978845608028654
                out_ref[:, :] = 0.5 * y * (1.0 + jnp.tanh(c * (y + 0.044715 * y * y * y)))

    return pl.pallas_call(
        body,
        out_shape=jax.ShapeDtypeStruct((chunk, n), jnp.float32),
        in_specs=[
            pl.BlockSpec(memory_space=pltpu.VMEM),
            pl.BlockSpec(memory_space=pltpu.VMEM),
        ],
        out_specs=pl.BlockSpec(memory_space=pltpu.VMEM),
        scratch_shapes=[
            pltpu.VMEM((N_DEV, chunk, n), jnp.float32),
            pltpu.SemaphoreType.DMA((N_DEV - 1,)),
            pltpu.SemaphoreType.DMA((N_DEV - 1,)),
        ],
        compiler_params=pltpu.CompilerParams(collective_id=0),
    )(x, w_mat)


# device time: 139962 ns/iter; 1.4748x vs baseline; 1.4748x over previous
import jax
import jax.numpy as jnp
from jax import lax
from jax.experimental import pallas as pl
from jax.experimental.pallas import tpu as pltpu

N_DEV = 16


def kernel(x, w_mat):
    m, _ = x.shape
    _, n = w_mat.shape
    chunk = m // N_DEV
    half = n // 2

    def body(x_ref, w_ref, out_ref,
             comm_a, comm_b, send_a, recv_a, send_b, recv_b):
        my = lax.axis_index("i")
        right = lax.rem(my + 1, N_DEV)
        left = lax.rem(my + N_DEV - 1, N_DEV)

        barrier = pltpu.get_barrier_semaphore()
        for nbr in (left, right):
            pl.semaphore_signal(
                barrier, inc=1,
                device_id=(nbr,), device_id_type=pl.DeviceIdType.MESH,
            )
        pl.semaphore_wait(barrier, 2)

        def partial_a(idx):
            x_sub = x_ref[pl.ds(idx * chunk, chunk), :]
            return jnp.dot(x_sub, w_ref[:, :half],
                           preferred_element_type=jnp.float32)

        def partial_b(idx):
            x_sub = x_ref[pl.ds(idx * chunk, chunk), :]
            return jnp.dot(x_sub, w_ref[:, half:],
                           preferred_element_type=jnp.float32)

        def gelu(y):
            c = 0.7978845608028654
            return 0.5 * y * (1.0 + jnp.tanh(c * (y + 0.044715 * y * y * y)))

        comm_a[0] = partial_a(lax.rem(my + N_DEV - 1, N_DEV))
        comm_b[0] = partial_b(lax.rem(my + 1, N_DEV))

        for s in range(N_DEV - 1):
            rdma_a = pltpu.make_async_remote_copy(
                src_ref=comm_a.at[s], dst_ref=comm_a.at[s + 1],
                send_sem=send_a.at[s], recv_sem=recv_a.at[s],
                device_id=(right,), device_id_type=pl.DeviceIdType.MESH,
            )
            rdma_b = pltpu.make_async_remote_copy(
                src_ref=comm_b.at[s], dst_ref=comm_b.at[s + 1],
                send_sem=send_b.at[s], recv_sem=recv_b.at[s],
                device_id=(left,), device_id_type=pl.DeviceIdType.MESH,
            )
            rdma_a.start()
            rdma_b.start()

            idx_a = lax.rem(my + (N_DEV - 2 - s), N_DEV)
            idx_b = lax.rem(my + s + 2, N_DEV)
            pa = partial_a(idx_a)
            pb = partial_b(idx_b)

            rdma_a.wait_recv()
            rdma_b.wait_recv()
            if s < N_DEV - 2:
                comm_a[s + 1] = comm_a[s + 1] + pa
                comm_b[s + 1] = comm_b[s + 1] + pb
            else:
                out_ref[:, :half] = gelu(comm_a[s + 1] + pa)
                out_ref[:, half:] = gelu(comm_b[s + 1] + pb)
            rdma_a.wait_send()
            rdma_b.wait_send()

    return pl.pallas_call(
        body,
        out_shape=jax.ShapeDtypeStruct((chunk, n), jnp.float32),
        in_specs=[
            pl.BlockSpec(memory_space=pltpu.VMEM),
            pl.BlockSpec(memory_space=pltpu.VMEM),
        ],
        out_specs=pl.BlockSpec(memory_space=pltpu.VMEM),
        scratch_shapes=[
            pltpu.VMEM((N_DEV, chunk, half), jnp.float32),
            pltpu.VMEM((N_DEV, chunk, half), jnp.float32),
            pltpu.SemaphoreType.DMA((N_DEV - 1,)),
            pltpu.SemaphoreType.DMA((N_DEV - 1,)),
            pltpu.SemaphoreType.DMA((N_DEV - 1,)),
            pltpu.SemaphoreType.DMA((N_DEV - 1,)),
        ],
        compiler_params=pltpu.CompilerParams(collective_id=0),
    )(x, w_mat)


# device time: 124233 ns/iter; 1.6616x vs baseline; 1.1266x over previous
import jax
import jax.numpy as jnp
from jax import lax
from jax.experimental import pallas as pl
from jax.experimental.pallas import tpu as pltpu

N_DEV = 16

PERM = [0, 4, 8, 12, 13, 9, 5, 1, 2, 6, 10, 14, 15, 11, 7, 3]
IPERM = [0] * N_DEV
for _p, _l in enumerate(PERM):
    IPERM[_l] = _p


def kernel(x, w_mat):
    m, _ = x.shape
    _, n = w_mat.shape
    chunk = m // N_DEV
    half = n // 2

    perm = jnp.array(PERM, dtype=jnp.int32)
    iperm = jnp.array(IPERM, dtype=jnp.int32)
    my = lax.axis_index("i")
    r = iperm[my]
    right = perm[jnp.mod(r + 1, N_DEV)]
    left = perm[jnp.mod(r - 1, N_DEV)]
    t = jnp.arange(N_DEV, dtype=jnp.int32)
    a_chunks = perm[jnp.mod(r - 1 - t, N_DEV)]
    b_chunks = perm[jnp.mod(r + 1 + t, N_DEV)]
    meta = jnp.concatenate(
        [right[None], left[None], a_chunks, b_chunks]
    ).astype(jnp.int32)

    def body(meta_ref, x_ref, w_ref, out_ref,
             comm_a, comm_b, send_a, recv_a, send_b, recv_b):
        right = meta_ref[0]
        left = meta_ref[1]

        barrier = pltpu.get_barrier_semaphore()
        for nbr in (left, right):
            pl.semaphore_signal(
                barrier, inc=1,
                device_id=(nbr,), device_id_type=pl.DeviceIdType.MESH,
            )
        pl.semaphore_wait(barrier, 2)

        def partial_a(idx):
            x_sub = x_ref[pl.ds(idx * chunk, chunk), :]
            return jnp.dot(x_sub, w_ref[:, :half],
                           preferred_element_type=jnp.float32)

        def partial_b(idx):
            x_sub = x_ref[pl.ds(idx * chunk, chunk), :]
            return jnp.dot(x_sub, w_ref[:, half:],
                           preferred_element_type=jnp.float32)

        def gelu(y):
            c = 0.7978845608028654
            return 0.5 * y * (1.0 + jnp.tanh(c * (y + 0.044715 * y * y * y)))

        comm_a[0] = partial_a(meta_ref[2])
        comm_b[0] = partial_b(meta_ref[2 + 16])

        for s in range(N_DEV - 1):
            rdma_a = pltpu.make_async_remote_copy(
                src_ref=comm_a.at[s], dst_ref=comm_a.at[s + 1],
                send_sem=send_a.at[s], recv_sem=recv_a.at[s],
                device_id=(right,), device_id_type=pl.DeviceIdType.MESH,
            )
            rdma_b = pltpu.make_async_remote_copy(
                src_ref=comm_b.at[s], dst_ref=comm_b.at[s + 1],
                send_sem=send_b.at[s], recv_sem=recv_b.at[s],
                device_id=(left,), device_id_type=pl.DeviceIdType.MESH,
            )
            rdma_a.start()
            rdma_b.start()

            pa = partial_a(meta_ref[2 + s + 1])
            pb = partial_b(meta_ref[2 + 16 + s + 1])

            rdma_a.wait_recv()
            rdma_b.wait_recv()
            if s < N_DEV - 2:
                comm_a[s + 1] = comm_a[s + 1] + pa
                comm_b[s + 1] = comm_b[s + 1] + pb
            else:
                out_ref[:, :half] = gelu(comm_a[s + 1] + pa)
                out_ref[:, half:] = gelu(comm_b[s + 1] + pb)
            rdma_a.wait_send()
            rdma_b.wait_send()

    return pl.pallas_call(
        body,
        out_shape=jax.ShapeDtypeStruct((chunk, n), jnp.float32),
        in_specs=[
            pl.BlockSpec(memory_space=pltpu.SMEM),
            pl.BlockSpec(memory_space=pltpu.VMEM),
            pl.BlockSpec(memory_space=pltpu.VMEM),
        ],
        out_specs=pl.BlockSpec(memory_space=pltpu.VMEM),
        scratch_shapes=[
            pltpu.VMEM((N_DEV, chunk, half), jnp.float32),
            pltpu.VMEM((N_DEV, chunk, half), jnp.float32),
            pltpu.SemaphoreType.DMA((N_DEV - 1,)),
            pltpu.SemaphoreType.DMA((N_DEV - 1,)),
            pltpu.SemaphoreType.DMA((N_DEV - 1,)),
            pltpu.SemaphoreType.DMA((N_DEV - 1,)),
        ],
        compiler_params=pltpu.CompilerParams(collective_id=0),
    )(meta, x, w_mat)


# device time: 98543 ns/iter; 2.0947x vs baseline; 1.2607x over previous
import jax
import jax.numpy as jnp
from jax import lax
from jax.experimental import pallas as pl
from jax.experimental.pallas import tpu as pltpu

N_DEV = 16
NSUB = 2

PERM = [0, 4, 8, 12, 13, 9, 5, 1, 2, 6, 10, 14, 15, 11, 7, 3]
IPERM = [0] * N_DEV
for _p, _l in enumerate(PERM):
    IPERM[_l] = _p


def kernel(x, w_mat):
    m, _ = x.shape
    _, n = w_mat.shape
    chunk = m // N_DEV
    half = n // 2
    sub = half // NSUB

    perm = jnp.array(PERM, dtype=jnp.int32)
    iperm = jnp.array(IPERM, dtype=jnp.int32)
    my = lax.axis_index("i")
    r = iperm[my]
    right = perm[jnp.mod(r + 1, N_DEV)]
    left = perm[jnp.mod(r - 1, N_DEV)]
    t = jnp.arange(N_DEV, dtype=jnp.int32)
    a_chunks = perm[jnp.mod(r - 1 - t, N_DEV)]
    b_chunks = perm[jnp.mod(r + 1 + t, N_DEV)]
    meta = jnp.concatenate(
        [right[None], left[None], a_chunks, b_chunks]
    ).astype(jnp.int32)

    def body(meta_ref, x_ref, w_ref, out_ref,
             comm_a, comm_b, send_a, recv_a, send_b, recv_b):
        right = meta_ref[0]
        left = meta_ref[1]

        def a_idx(tt):
            return meta_ref[2 + tt]

        def b_idx(tt):
            return meta_ref[2 + N_DEV + tt]

        def partial_a(idx):
            x_sub = x_ref[pl.ds(idx * chunk, chunk), :]
            return jnp.dot(x_sub, w_ref[:, :half],
                           preferred_element_type=jnp.float32)

        def partial_b(idx):
            x_sub = x_ref[pl.ds(idx * chunk, chunk), :]
            return jnp.dot(x_sub, w_ref[:, half:],
                           preferred_element_type=jnp.float32)

        def gelu(y):
            c = 0.7978845608028654
            return 0.5 * y * (1.0 + jnp.tanh(c * (y + 0.044715 * y * y * y)))

        def make(dir_comm, dir_send, dir_recv, slot, j, nbr):
            return pltpu.make_async_remote_copy(
                src_ref=dir_comm.at[slot, j],
                dst_ref=dir_comm.at[slot + 1, j],
                send_sem=dir_send.at[slot, j],
                recv_sem=dir_recv.at[slot, j],
                device_id=(nbr,), device_id_type=pl.DeviceIdType.MESH,
            )

        barrier = pltpu.get_barrier_semaphore()
        for nbr in (left, right):
            pl.semaphore_signal(
                barrier, inc=1,
                device_id=(nbr,), device_id_type=pl.DeviceIdType.MESH,
            )

        pa = partial_a(a_idx(0))
        pb = partial_b(b_idx(0))
        pl.semaphore_wait(barrier, 2)

        for tt in range(N_DEV):
            if tt == 0:
                for j in range(NSUB):
                    comm_a[0, j] = pa[:, j * sub:(j + 1) * sub]
                    comm_b[0, j] = pb[:, j * sub:(j + 1) * sub]
                    make(comm_a, send_a, recv_a, 0, j, right).start()
                    make(comm_b, send_b, recv_b, 0, j, left).start()
            else:
                for j in range(NSUB):
                    for (comm, s_sem, r_sem, nbr, p) in (
                        (comm_a, send_a, recv_a, right, pa),
                        (comm_b, send_b, recv_b, left, pb),
                    ):
                        rd = make(comm, s_sem, r_sem, tt - 1, j, nbr)
                        rd.wait_recv()
                        acc = comm[tt, j] + p[:, j * sub:(j + 1) * sub]
                        if tt < N_DEV - 1:
                            comm[tt, j] = acc
                            make(comm, s_sem, r_sem, tt, j, nbr).start()
                        else:
                            base = (0 if comm is comm_a else half) + j * sub
                            out_ref[:, base:base + sub] = gelu(acc)
                for j in range(NSUB):
                    make(comm_a, send_a, recv_a, tt - 1, j, right).wait_send()
                    make(comm_b, send_b, recv_b, tt - 1, j, left).wait_send()
            if tt < N_DEV - 1:
                pa = partial_a(a_idx(tt + 1))
                pb = partial_b(b_idx(tt + 1))

    return pl.pallas_call(
        body,
        out_shape=jax.ShapeDtypeStruct((chunk, n), jnp.float32),
        in_specs=[
            pl.BlockSpec(memory_space=pltpu.SMEM),
            pl.BlockSpec(memory_space=pltpu.VMEM),
            pl.BlockSpec(memory_space=pltpu.VMEM),
        ],
        out_specs=pl.BlockSpec(memory_space=pltpu.VMEM),
        scratch_shapes=[
            pltpu.VMEM((N_DEV, NSUB, chunk, sub), jnp.float32),
            pltpu.VMEM((N_DEV, NSUB, chunk, sub), jnp.float32),
            pltpu.SemaphoreType.DMA((N_DEV - 1, NSUB)),
            pltpu.SemaphoreType.DMA((N_DEV - 1, NSUB)),
            pltpu.SemaphoreType.DMA((N_DEV - 1, NSUB)),
            pltpu.SemaphoreType.DMA((N_DEV - 1, NSUB)),
        ],
        compiler_params=pltpu.CompilerParams(collective_id=0),
    )(meta, x, w_mat)


# device time: 61418 ns/iter; 3.3609x vs baseline; 1.6045x over previous
import jax
import jax.numpy as jnp
from jax import lax
from jax.experimental import pallas as pl
from jax.experimental.pallas import tpu as pltpu

N_DEV = 16
NSUB = 2

PERM = [0, 4, 8, 12, 13, 9, 5, 1, 2, 6, 10, 14, 15, 11, 7, 3]
IPERM = [0] * N_DEV
for _p, _l in enumerate(PERM):
    IPERM[_l] = _p


def kernel(x, w_mat):
    m, _ = x.shape
    _, n = w_mat.shape
    chunk = m // N_DEV
    half = n // 2
    sub = half // NSUB

    perm = jnp.array(PERM, dtype=jnp.int32)
    iperm = jnp.array(IPERM, dtype=jnp.int32)
    my = lax.axis_index("i")
    r = iperm[my]
    right = perm[jnp.mod(r + 1, N_DEV)]
    left = perm[jnp.mod(r - 1, N_DEV)]
    t = jnp.arange(N_DEV, dtype=jnp.int32)
    a_chunks = perm[jnp.mod(r - 1 - t, N_DEV)]
    b_chunks = perm[jnp.mod(r + 1 + t, N_DEV)]
    meta = jnp.concatenate(
        [right[None], left[None], a_chunks, b_chunks]
    ).astype(jnp.int32)

    def body(meta_ref, x_ref, w_ref, out_ref,
             comm_a, comm_b, send_a, recv_a, send_b, recv_b):
        right = meta_ref[0]
        left = meta_ref[1]

        def a_idx(tt):
            return meta_ref[2 + tt]

        def b_idx(tt):
            return meta_ref[2 + N_DEV + tt]

        def partial_a(idx):
            x_sub = x_ref[pl.ds(idx * chunk, chunk), :]
            return jnp.dot(x_sub, w_ref[:, :half],
                           preferred_element_type=jnp.float32)

        def partial_b(idx):
            x_sub = x_ref[pl.ds(idx * chunk, chunk), :]
            return jnp.dot(x_sub, w_ref[:, half:],
                           preferred_element_type=jnp.float32)

        def gelu(y):
            c = 0.7978845608028654
            return 0.5 * y * (1.0 + jnp.tanh(c * (y + 0.044715 * y * y * y)))

        def make(dir_comm, dir_send, dir_recv, slot, j, nbr):
            return pltpu.make_async_remote_copy(
                src_ref=dir_comm.at[slot, j],
                dst_ref=dir_comm.at[slot + 1, j],
                send_sem=dir_send.at[slot, j],
                recv_sem=dir_recv.at[slot, j],
                device_id=(nbr,), device_id_type=pl.DeviceIdType.MESH,
            )

        barrier = pltpu.get_barrier_semaphore()
        for nbr in (left, right):
            pl.semaphore_signal(
                barrier, inc=1,
                device_id=(nbr,), device_id_type=pl.DeviceIdType.MESH,
            )

        pa = partial_a(a_idx(0))
        pb = partial_b(b_idx(0))
        pl.semaphore_wait(barrier, 2)

        for tt in range(N_DEV):
            if tt == 0:
                for j in range(NSUB):
                    comm_a[0, j] = pa[:, j * sub:(j + 1) * sub].astype(jnp.bfloat16)
                    comm_b[0, j] = pb[:, j * sub:(j + 1) * sub].astype(jnp.bfloat16)
                    make(comm_a, send_a, recv_a, 0, j, right).start()
                    make(comm_b, send_b, recv_b, 0, j, left).start()
            else:
                for j in range(NSUB):
                    for (comm, s_sem, r_sem, nbr, p) in (
                        (comm_a, send_a, recv_a, right, pa),
                        (comm_b, send_b, recv_b, left, pb),
                    ):
                        rd = make(comm, s_sem, r_sem, tt - 1, j, nbr)
                        rd.wait_recv()
                        acc = (comm[tt, j].astype(jnp.float32)
                               + p[:, j * sub:(j + 1) * sub])
                        if tt < N_DEV - 1:
                            comm[tt, j] = acc.astype(jnp.bfloat16)
                            make(comm, s_sem, r_sem, tt, j, nbr).start()
                        else:
                            base = (0 if comm is comm_a else half) + j * sub
                            out_ref[:, base:base + sub] = gelu(acc)
                for j in range(NSUB):
                    make(comm_a, send_a, recv_a, tt - 1, j, right).wait_send()
                    make(comm_b, send_b, recv_b, tt - 1, j, left).wait_send()
            if tt < N_DEV - 1:
                pa = partial_a(a_idx(tt + 1))
                pb = partial_b(b_idx(tt + 1))

    return pl.pallas_call(
        body,
        out_shape=jax.ShapeDtypeStruct((chunk, n), jnp.float32),
        in_specs=[
            pl.BlockSpec(memory_space=pltpu.SMEM),
            pl.BlockSpec(memory_space=pltpu.VMEM),
            pl.BlockSpec(memory_space=pltpu.VMEM),
        ],
        out_specs=pl.BlockSpec(memory_space=pltpu.VMEM),
        scratch_shapes=[
            pltpu.VMEM((N_DEV, NSUB, chunk, sub), jnp.bfloat16),
            pltpu.VMEM((N_DEV, NSUB, chunk, sub), jnp.bfloat16),
            pltpu.SemaphoreType.DMA((N_DEV - 1, NSUB)),
            pltpu.SemaphoreType.DMA((N_DEV - 1, NSUB)),
            pltpu.SemaphoreType.DMA((N_DEV - 1, NSUB)),
            pltpu.SemaphoreType.DMA((N_DEV - 1, NSUB)),
        ],
        compiler_params=pltpu.CompilerParams(collective_id=0),
    )(meta, x, w_mat)


# device time: 57764 ns/iter; 3.5735x vs baseline; 1.0633x over previous
import jax
import jax.numpy as jnp
from jax import lax
from jax.experimental import pallas as pl
from jax.experimental.pallas import tpu as pltpu

N_DEV = 16
NSUB = 4

PERM = [0, 4, 8, 12, 13, 9, 5, 1, 2, 6, 10, 14, 15, 11, 7, 3]
IPERM = [0] * N_DEV
for _p, _l in enumerate(PERM):
    IPERM[_l] = _p


def kernel(x, w_mat):
    m, _ = x.shape
    _, n = w_mat.shape
    chunk = m // N_DEV
    half = n // 2
    sub = half // NSUB

    perm = jnp.array(PERM, dtype=jnp.int32)
    iperm = jnp.array(IPERM, dtype=jnp.int32)
    my = lax.axis_index("i")
    r = iperm[my]
    right = perm[jnp.mod(r + 1, N_DEV)]
    left = perm[jnp.mod(r - 1, N_DEV)]
    t = jnp.arange(N_DEV, dtype=jnp.int32)
    a_chunks = perm[jnp.mod(r - 1 - t, N_DEV)]
    b_chunks = perm[jnp.mod(r + 1 + t, N_DEV)]
    meta = jnp.concatenate(
        [right[None], left[None], a_chunks, b_chunks]
    ).astype(jnp.int32)

    def body(meta_ref, x_ref, w_ref, out_ref,
             comm_a, comm_b, send_a, recv_a, send_b, recv_b):
        right = meta_ref[0]
        left = meta_ref[1]

        def a_idx(tt):
            return meta_ref[2 + tt]

        def b_idx(tt):
            return meta_ref[2 + N_DEV + tt]

        def partial_a(idx):
            x_sub = x_ref[pl.ds(idx * chunk, chunk), :]
            return jnp.dot(x_sub, w_ref[:, :half],
                           preferred_element_type=jnp.float32)

        def partial_b(idx):
            x_sub = x_ref[pl.ds(idx * chunk, chunk), :]
            return jnp.dot(x_sub, w_ref[:, half:],
                           preferred_element_type=jnp.float32)

        def gelu(y):
            c = 0.7978845608028654
            return 0.5 * y * (1.0 + jnp.tanh(c * (y + 0.044715 * y * y * y)))

        def make(dir_comm, dir_send, dir_recv, slot, j, nbr):
            return pltpu.make_async_remote_copy(
                src_ref=dir_comm.at[slot, j],
                dst_ref=dir_comm.at[slot + 1, j],
                send_sem=dir_send.at[slot, j],
                recv_sem=dir_recv.at[slot, j],
                device_id=(nbr,), device_id_type=pl.DeviceIdType.MESH,
            )

        barrier = pltpu.get_barrier_semaphore()
        for nbr in (left, right):
            pl.semaphore_signal(
                barrier, inc=1,
                device_id=(nbr,), device_id_type=pl.DeviceIdType.MESH,
            )

        pa = partial_a(a_idx(0))
        pb = partial_b(b_idx(0))
        pl.semaphore_wait(barrier, 2)

        for tt in range(N_DEV):
            if tt == 0:
                for j in range(NSUB):
                    comm_a[0, j] = pa[:, j * sub:(j + 1) * sub].astype(jnp.bfloat16)
                    comm_b[0, j] = pb[:, j * sub:(j + 1) * sub].astype(jnp.bfloat16)
                    make(comm_a, send_a, recv_a, 0, j, right).start()
                    make(comm_b, send_b, recv_b, 0, j, left).start()
            else:
                for j in range(NSUB):
                    for (comm, s_sem, r_sem, nbr, p) in (
                        (comm_a, send_a, recv_a, right, pa),
                        (comm_b, send_b, recv_b, left, pb),
                    ):
                        rd = make(comm, s_sem, r_sem, tt - 1, j, nbr)
                        rd.wait_recv()
                        acc = (comm[tt, j].astype(jnp.float32)
                               + p[:, j * sub:(j + 1) * sub])
                        if tt < N_DEV - 1:
                            comm[tt, j] = acc.astype(jnp.bfloat16)
                            make(comm, s_sem, r_sem, tt, j, nbr).start()
                        else:
                            base = (0 if comm is comm_a else half) + j * sub
                            out_ref[:, base:base + sub] = gelu(acc)
                for j in range(NSUB):
                    make(comm_a, send_a, recv_a, tt - 1, j, right).wait_send()
                    make(comm_b, send_b, recv_b, tt - 1, j, left).wait_send()
            if tt < N_DEV - 1:
                pa = partial_a(a_idx(tt + 1))
                pb = partial_b(b_idx(tt + 1))

    return pl.pallas_call(
        body,
        out_shape=jax.ShapeDtypeStruct((chunk, n), jnp.float32),
        in_specs=[
            pl.BlockSpec(memory_space=pltpu.SMEM),
            pl.BlockSpec(memory_space=pltpu.VMEM),
            pl.BlockSpec(memory_space=pltpu.VMEM),
        ],
        out_specs=pl.BlockSpec(memory_space=pltpu.VMEM),
        scratch_shapes=[
            pltpu.VMEM((N_DEV, NSUB, chunk, sub), jnp.bfloat16),
            pltpu.VMEM((N_DEV, NSUB, chunk, sub), jnp.bfloat16),
            pltpu.SemaphoreType.DMA((N_DEV - 1, NSUB)),
            pltpu.SemaphoreType.DMA((N_DEV - 1, NSUB)),
            pltpu.SemaphoreType.DMA((N_DEV - 1, NSUB)),
            pltpu.SemaphoreType.DMA((N_DEV - 1, NSUB)),
        ],
        compiler_params=pltpu.CompilerParams(collective_id=0),
    )(meta, x, w_mat)
